# baseline (speedup 1.0000x reference)
import jax
import jax.numpy as jnp
from jax import lax
from jax.experimental import pallas as pl
from jax.experimental.pallas import tpu as pltpu
from jax.experimental.pallas import tpu_sc as plsc

_ED = 64
_PD = 16
_NW = 32
_CHUNK = 128
_NBUF = 4


def _make_sc_gather(n_total, bsz):
    per_w = n_total // _NW
    nch = per_w // _CHUNK
    ch_per_h = bsz // _CHUNK
    mesh = plsc.VectorSubcoreMesh(core_axis_name="c", subcore_axis_name="s")

    def body(x_hbm, cat_hbm, real_out, imag_out, *scr):
        bufc = scr[0:_NBUF]
        tbr = scr[_NBUF:2 * _NBUF]
        tbi = scr[2 * _NBUF:3 * _NBUF]
        idxr = scr[3 * _NBUF]
        gsem = scr[3 * _NBUF + 1:4 * _NBUF + 1]
        osem = scr[4 * _NBUF + 1:5 * _NBUF + 1]

        info = plsc.get_sparse_core_info()
        wid = lax.axis_index("s") * info.num_cores + lax.axis_index("c")
        iota = lax.iota(jnp.int32, 16)

        def load_idx(j, b):
            pltpu.sync_copy(x_hbm.at[pl.ds(wid * nch + j, 1), :],
                            idxr.at[pl.ds(b, 1), :])

        def fire_gather(b):
            pltpu.async_copy(cat_hbm.at[idxr.at[b]], bufc[b], gsem[b])

        def drain_gather(b):
            pltpu.make_async_copy(cat_hbm.at[pl.ds(0, _CHUNK)], bufc[b],
                                  gsem[b]).wait()

        def transpose(b):
            def diag(d, carry):
                rot = (iota + d) & 15
                for ib in range(_CHUNK // 16):
                    ivec = ib * 16 + iota
                    for eb in range(_ED // 16):
                        g = plsc.load_gather(bufc[b], [ivec, eb * 16 + rot])
                        plsc.store_scatter(tbr[b], [eb * 16 + rot, ivec], g)
                    g = plsc.load_gather(bufc[b], [ivec, _ED + rot])
                    plsc.store_scatter(tbi[b], [rot, ivec], g)
                return carry
            lax.fori_loop(0, 16, diag, 0)

        def fire_out(j, b):
            c = wid * nch + j
            h = c // ch_per_h
            bt = c % ch_per_h
            for et in range(_ED // 8):
                row = ((h * (_ED // 8) + et) * ch_per_h + bt) * 8
                pltpu.async_copy(tbr[b].at[pl.ds(et * 8, 8), :],
                                 real_out.at[pl.ds(row, 8), :], osem[b])
            for et in range(_PD // 8):
                row = ((h * (_PD // 8) + et) * ch_per_h + bt) * 8
                pltpu.async_copy(tbi[b].at[pl.ds(et * 8, 8), :],
                                 imag_out.at[pl.ds(row, 8), :], osem[b])

        def drain_out(b):
            pltpu.make_async_copy(tbr[b], real_out.at[pl.ds(0, _ED), :],
                                  osem[b]).wait()
            pltpu.make_async_copy(tbi[b], imag_out.at[pl.ds(0, _PD), :],
                                  osem[b]).wait()

        for b in range(_NBUF):
            load_idx(b, b)
            fire_gather(b)

        def outer(j0, carry):
            for b in range(_NBUF):
                j = j0 * _NBUF + b
                drain_gather(b)

                @pl.when(j >= _NBUF)
                def _():
                    drain_out(b)

                transpose(b)
                fire_out(j, b)

                @pl.when(j + _NBUF < nch)
                def _():
                    load_idx(j + _NBUF, b)
                    fire_gather(b)
            return carry

        lax.fori_loop(0, nch // _NBUF, outer, 0)
        for b in range(_NBUF):
            drain_out(b)

    return pl.kernel(
        body,
        out_type=(
            jax.ShapeDtypeStruct((n_total * _ED // 128, 128), jnp.float32),
            jax.ShapeDtypeStruct((n_total * _PD // 128, 128), jnp.float32),
        ),
        mesh=mesh,
        scratch_types=(
            [pltpu.VMEM((_CHUNK, _ED + _PD), jnp.float32)] * _NBUF
            + [pltpu.VMEM((_ED, _CHUNK), jnp.float32)] * _NBUF
            + [pltpu.VMEM((_PD, _CHUNK), jnp.float32)] * _NBUF
            + [pltpu.VMEM((_NBUF, _CHUNK), jnp.int32)]
            + [pltpu.SemaphoreType.DMA] * (2 * _NBUF)
        ),
        compiler_params=pltpu.CompilerParams(use_tc_tiling_on_sc=False,
                                             needs_layout_passes=False),
    )


def kernel(x, real_table, imag_table):
    bsz, hist = x.shape
    n = bsz * hist
    vocab = real_table.shape[0]
    xt = jnp.transpose(x).reshape(n // _CHUNK, _CHUNK).astype(jnp.int32)
    cat = jnp.concatenate([real_table, imag_table], axis=1)
    r1, i1 = _make_sc_gather(n, bsz)(xt, cat)
    real = (r1.reshape(hist, _ED // 8, bsz // _CHUNK, 8, _CHUNK)
            .transpose(2, 4, 0, 1, 3).reshape(bsz, hist, _ED))
    imag = (i1.reshape(hist, _PD // 8, bsz // _CHUNK, 8, _CHUNK)
            .transpose(2, 4, 0, 1, 3).reshape(bsz, hist, _PD))
    return (real, imag)


# speedup vs baseline: 1.2739x; 1.2739x over previous
import jax
import jax.numpy as jnp
from jax import lax
from jax.experimental import pallas as pl
from jax.experimental.pallas import tpu as pltpu
from jax.experimental.pallas import tpu_sc as plsc

_ED = 64
_PD = 16
_NW = 32
_CHUNK = 128
_NBUF = 5


def _make_sc_gather(n_total, bsz):
    per_w = n_total // _NW
    nch = per_w // _CHUNK
    ch_per_h = bsz // _CHUNK
    mesh = plsc.VectorSubcoreMesh(core_axis_name="c", subcore_axis_name="s")

    def body(x_hbm, real_hbm, imag_hbm, real_out, imag_out, *scr):
        bufr = scr[0:_NBUF]
        bufi = scr[_NBUF:2 * _NBUF]
        tbr = scr[2 * _NBUF:3 * _NBUF]
        tbi = scr[3 * _NBUF:4 * _NBUF]
        idxr = scr[4 * _NBUF]
        gsem = scr[4 * _NBUF + 1:5 * _NBUF + 1]
        osem = scr[5 * _NBUF + 1:6 * _NBUF + 1]

        info = plsc.get_sparse_core_info()
        wid = lax.axis_index("s") * info.num_cores + lax.axis_index("c")
        iota = lax.iota(jnp.int32, 16)

        def load_idx(j, b):
            pltpu.sync_copy(x_hbm.at[pl.ds(wid * nch + j, 1), :],
                            idxr.at[pl.ds(b, 1), :])

        def fire_gather(b):
            pltpu.async_copy(real_hbm.at[idxr.at[b]], bufr[b], gsem[b])
            pltpu.async_copy(imag_hbm.at[idxr.at[b]], bufi[b], gsem[b])

        def drain_gather(b):
            pltpu.make_async_copy(real_hbm.at[pl.ds(0, _CHUNK)], bufr[b],
                                  gsem[b]).wait()
            pltpu.make_async_copy(imag_hbm.at[pl.ds(0, _CHUNK)], bufi[b],
                                  gsem[b]).wait()

        def transpose(b):
            def diag(d, carry):
                rot = (iota + d) & 15
                for ib in range(_CHUNK // 16):
                    ivec = ib * 16 + iota
                    for eb in range(_ED // 16):
                        g = plsc.load_gather(bufr[b], [ivec, eb * 16 + rot])
                        plsc.store_scatter(tbr[b], [eb * 16 + rot, ivec], g)
                    g = plsc.load_gather(bufi[b], [ivec, rot])
                    plsc.store_scatter(tbi[b], [rot, ivec], g)
                return carry
            lax.fori_loop(0, 16, diag, 0)

        def fire_out(j, b):
            c = wid * nch + j
            h = c // ch_per_h
            bt = c % ch_per_h
            for et in range(_ED // 8):
                row = ((h * (_ED // 8) + et) * ch_per_h + bt) * 8
                pltpu.async_copy(tbr[b].at[pl.ds(et * 8, 8), :],
                                 real_out.at[pl.ds(row, 8), :], osem[b])
            for et in range(_PD // 8):
                row = ((h * (_PD // 8) + et) * ch_per_h + bt) * 8
                pltpu.async_copy(tbi[b].at[pl.ds(et * 8, 8), :],
                                 imag_out.at[pl.ds(row, 8), :], osem[b])

        def drain_out(b):
            pltpu.make_async_copy(tbr[b], real_out.at[pl.ds(0, _ED), :],
                                  osem[b]).wait()
            pltpu.make_async_copy(tbi[b], imag_out.at[pl.ds(0, _PD), :],
                                  osem[b]).wait()

        for b in range(_NBUF):
            load_idx(b, b)
            fire_gather(b)

        def outer(j0, carry):
            for b in range(_NBUF):
                j = j0 * _NBUF + b
                drain_gather(b)

                @pl.when(j >= _NBUF)
                def _():
                    drain_out(b)

                transpose(b)
                fire_out(j, b)

                @pl.when(j + _NBUF < nch)
                def _():
                    load_idx(j + _NBUF, b)
                    fire_gather(b)
            return carry

        lax.fori_loop(0, nch // _NBUF, outer, 0)
        for b in range(_NBUF):
            drain_out(b)

    return pl.kernel(
        body,
        out_type=(
            jax.ShapeDtypeStruct((n_total * _ED // 128, 128), jnp.float32),
            jax.ShapeDtypeStruct((n_total * _PD // 128, 128), jnp.float32),
        ),
        mesh=mesh,
        scratch_types=(
            [pltpu.VMEM((_CHUNK, _ED), jnp.float32)] * _NBUF
            + [pltpu.VMEM((_CHUNK, _PD), jnp.float32)] * _NBUF
            + [pltpu.VMEM((_ED, _CHUNK), jnp.float32)] * _NBUF
            + [pltpu.VMEM((_PD, _CHUNK), jnp.float32)] * _NBUF
            + [pltpu.VMEM((_NBUF, _CHUNK), jnp.int32)]
            + [pltpu.SemaphoreType.DMA] * (2 * _NBUF)
        ),
        compiler_params=pltpu.CompilerParams(use_tc_tiling_on_sc=False,
                                             needs_layout_passes=False),
    )


def kernel(x, real_table, imag_table):
    bsz, hist = x.shape
    n = bsz * hist
    vocab = real_table.shape[0]
    xt = jnp.transpose(x).reshape(n // _CHUNK, _CHUNK).astype(jnp.int32)
    rt = real_table
    it = imag_table
    r1, i1 = _make_sc_gather(n, bsz)(xt, rt, it)
    real = (r1.reshape(hist, _ED // 8, bsz // _CHUNK, 8, _CHUNK)
            .transpose(2, 4, 0, 1, 3).reshape(bsz, hist, _ED))
    imag = (i1.reshape(hist, _PD // 8, bsz // _CHUNK, 8, _CHUNK)
            .transpose(2, 4, 0, 1, 3).reshape(bsz, hist, _PD))
    return (real, imag)
